# baseline (speedup 1.0000x reference)
import functools

import jax
import jax.numpy as jnp
from jax import lax
from jax.experimental import pallas as pl
from jax.experimental.pallas import tpu as pltpu
from jax.experimental.pallas import tpu_sc as plsc

_NUM_FIELDS = 26
_VOCAB = 100000
_EMBED_DIM = 16
_NUM_DENSE = 13

_NUM_CORES = 2
_NUM_SUBCORES = 16
_NUM_WORKERS = _NUM_CORES * _NUM_SUBCORES


@functools.partial(jax.jit, static_argnames=("batch", "chunk"))
def _embed_cat(idx_fm, tables_flat, *, batch, chunk):
    rows_per_w = batch // _NUM_WORKERS
    n_chunks = rows_per_w // chunk
    mesh = plsc.VectorSubcoreMesh(
        core_axis_name="c", subcore_axis_name="s")

    @functools.partial(
        pl.kernel,
        out_type=jax.ShapeDtypeStruct((_NUM_FIELDS, batch, _EMBED_DIM),
                                      jnp.float32),
        mesh=mesh,
        compiler_params=pltpu.CompilerParams(use_tc_tiling_on_sc=False),
        scratch_types=[
            pltpu.VMEM((_NUM_FIELDS * rows_per_w,), jnp.int32),
            pltpu.VMEM((chunk, _EMBED_DIM), jnp.float32),
            pltpu.VMEM((chunk, _EMBED_DIM), jnp.float32),
            pltpu.SemaphoreType.DMA,
            pltpu.SemaphoreType.DMA,
            pltpu.SemaphoreType.DMA,
        ],
    )
    def k(idx_hbm, tab_hbm, out_hbm, idx_v, emb_a, emb_b, sem_i, sem_g,
          sem_o):
        wid = lax.axis_index("s") * _NUM_CORES + lax.axis_index("c")
        wbase = wid * rows_per_w

        icps = []
        for f in range(_NUM_FIELDS):
            icps.append(pltpu.async_copy(
                idx_hbm.at[pl.ds(f * batch + wbase, rows_per_w)],
                idx_v.at[pl.ds(f * rows_per_w, rows_per_w)],
                sem_i,
            ))
        for cp in icps:
            cp.wait()

        bufs = (emb_a, emb_b)

        def step_body(step, _):
            f = step // n_chunks
            ci = step % n_chunks
            buf = step % 2

            def run(emb_v):
                gcp = pltpu.async_copy(
                    tab_hbm.at[idx_v.at[pl.ds(f * rows_per_w + ci * chunk,
                                              chunk)]],
                    emb_v,
                    sem_g,
                )
                gcp.wait()
                pltpu.sync_copy(
                    emb_v,
                    out_hbm.at[f, pl.ds(wbase + ci * chunk, chunk)],
                )

            lax.cond(buf == 0, lambda: run(bufs[0]), lambda: run(bufs[1]))
            return 0

        lax.fori_loop(0, _NUM_FIELDS * n_chunks, step_body, 0)

    return k(idx_fm, tables_flat)


def kernel(categorical, numerical, tables):
    batch = categorical.shape[0]
    idx_fm = (categorical.astype(jnp.int32).T
              + (jnp.arange(_NUM_FIELDS, dtype=jnp.int32) * _VOCAB)[:, None]
              ).reshape(_NUM_FIELDS * batch)
    tables_flat = tables.reshape(_NUM_FIELDS * _VOCAB, _EMBED_DIM)
    emb = _embed_cat(idx_fm, tables_flat, batch=batch, chunk=128)
    emb_bf = emb.transpose(1, 0, 2).reshape(batch,
                                            _NUM_FIELDS * _EMBED_DIM)
    return jnp.concatenate([numerical, emb_bf], axis=-1)


# speedup vs baseline: 1.1448x; 1.1448x over previous
import functools

import jax
import jax.numpy as jnp
from jax import lax
from jax.experimental import pallas as pl
from jax.experimental.pallas import tpu as pltpu
from jax.experimental.pallas import tpu_sc as plsc

_NUM_FIELDS = 26
_VOCAB = 100000
_EMBED_DIM = 16
_NUM_DENSE = 13

_NUM_CORES = 2
_NUM_SUBCORES = 16
_NUM_WORKERS = _NUM_CORES * _NUM_SUBCORES

_GATHER = 128
_GPS = 13
_SUPER = _GATHER * _GPS


@functools.partial(jax.jit, static_argnames=("batch",))
def _embed_flat(idx_flat, tables_flat, *, batch):
    total = batch * _NUM_FIELDS
    per_w = total // _NUM_WORKERS
    n_super = per_w // _SUPER
    mesh = plsc.VectorSubcoreMesh(
        core_axis_name="c", subcore_axis_name="s")

    @functools.partial(
        pl.kernel,
        out_type=jax.ShapeDtypeStruct((total, _EMBED_DIM), jnp.float32),
        mesh=mesh,
        compiler_params=pltpu.CompilerParams(use_tc_tiling_on_sc=False),
        scratch_types=[
            pltpu.VMEM((per_w,), jnp.int32),
            pltpu.VMEM((_SUPER, _EMBED_DIM), jnp.float32),
            pltpu.VMEM((_SUPER, _EMBED_DIM), jnp.float32),
            pltpu.SemaphoreType.DMA,
            pltpu.SemaphoreType.DMA,
            pltpu.SemaphoreType.DMA,
        ],
    )
    def k(idx_hbm, tab_hbm, out_hbm, idx_v, buf_a, buf_b, sem_i, sem_g,
          sem_o):
        wid = lax.axis_index("s") * _NUM_CORES + lax.axis_index("c")
        wbase = wid * per_w

        pltpu.async_copy(
            idx_hbm.at[pl.ds(wbase, per_w)], idx_v, sem_i,
        ).wait()

        bufs = (buf_a, buf_b)

        def super_body(s, _):
            sbase = s * _SUPER

            def run(buf):
                gcps = []
                for j in range(_GPS):
                    gcps.append(pltpu.async_copy(
                        tab_hbm.at[idx_v.at[pl.ds(sbase + j * _GATHER,
                                                  _GATHER)]],
                        buf.at[pl.ds(j * _GATHER, _GATHER)],
                        sem_g,
                    ))
                for cp in gcps:
                    cp.wait()
                pltpu.async_copy(
                    buf, out_hbm.at[pl.ds(wbase + sbase, _SUPER)], sem_o,
                ).wait()

            lax.cond(s % 2 == 0, lambda: run(bufs[0]),
                     lambda: run(bufs[1]))
            return 0

        lax.fori_loop(0, n_super, super_body, 0)

    return k(idx_flat, tables_flat)


def kernel(categorical, numerical, tables):
    batch = categorical.shape[0]
    idx_flat = (categorical.astype(jnp.int32)
                + (jnp.arange(_NUM_FIELDS, dtype=jnp.int32)
                   * _VOCAB)[None, :]).reshape(batch * _NUM_FIELDS)
    tables_flat = tables.reshape(_NUM_FIELDS * _VOCAB, _EMBED_DIM)
    emb = _embed_flat(idx_flat, tables_flat, batch=batch)
    emb_bf = emb.reshape(batch, _NUM_FIELDS * _EMBED_DIM)
    return jnp.concatenate([numerical, emb_bf], axis=-1)
